# baseline (speedup 1.0000x reference)
import functools

import jax
import jax.numpy as jnp
from jax import lax
from jax.experimental import pallas as pl
from jax.experimental.pallas import tpu as pltpu
from jax.experimental.pallas import tpu_sc as plsc

B, N, C = 64, 1024, 192
K = N // 4
NT = N + 1
JC = 128


def _rank_body(x_ref, ea_ref, idx_ref):
    b = pl.program_id(0)
    nodes = x_ref[0, 1:, :]
    ea = ea_ref[0]
    m = jnp.sum(ea, axis=0, keepdims=True) * (1.0 / N)
    ci = jax.nn.sigmoid(m)
    scores = jnp.dot(nodes, jnp.reshape(ci, (C, 1)),
                     preferred_element_type=jnp.float32)
    s_row = jnp.reshape(scores, (1, N))

    jj0 = lax.broadcasted_iota(jnp.int32, (JC, N), 0)
    ii = lax.broadcasted_iota(jnp.int32, (JC, N), 1)

    rank_row = jnp.zeros((1, N), jnp.int32)
    for jc in range(N // JC):
        s_col = scores[jc * JC:(jc + 1) * JC, :]
        jj = jj0 + (jc * JC)
        before = (s_col < s_row) | ((s_col == s_row) & (jj < ii))
        rank_row = rank_row + jnp.sum(before.astype(jnp.int32), axis=0,
                                      keepdims=True)
    ranks = jnp.reshape(rank_row, (N, 1))

    p_iota = lax.broadcasted_iota(jnp.int32, (N, K), 1) + (N - K)
    ivals = lax.broadcasted_iota(jnp.int32, (N, 1), 0) + (b * NT + 1)
    idx_row = jnp.sum(jnp.where(ranks == p_iota, ivals, 0), axis=0,
                      keepdims=True)
    idx_ref[0] = idx_row


_mesh = plsc.VectorSubcoreMesh(core_axis_name="c", subcore_axis_name="s")


@functools.partial(
    pl.kernel,
    out_type=jax.ShapeDtypeStruct((B, K, C), jnp.float32),
    mesh=_mesh,
    scratch_types=[
        pltpu.VMEM((K,), jnp.int32),
        pltpu.VMEM((K, C), jnp.float32),
        pltpu.SemaphoreType.DMA,
    ],
    compiler_params=pltpu.CompilerParams(use_tc_tiling_on_sc=False),
)
def _sc_gather(xflat, idx_hbm, out_hbm, idx_v, rows_v, sem):
    wid = lax.axis_index("s") * 2 + lax.axis_index("c")
    for j in range(2):
        b = wid * 2 + j
        pltpu.sync_copy(idx_hbm.at[b], idx_v)
        cp0 = pltpu.async_copy(xflat.at[idx_v.at[pl.ds(0, 128)]],
                               rows_v.at[pl.ds(0, 128)], sem)
        cp1 = pltpu.async_copy(xflat.at[idx_v.at[pl.ds(128, 128)]],
                               rows_v.at[pl.ds(128, 128)], sem)
        cp0.wait()
        cp1.wait()
        pltpu.sync_copy(rows_v, out_hbm.at[b])


@jax.jit
def kernel(x, edge_aggregation):
    idx = pl.pallas_call(
        _rank_body,
        grid=(B,),
        in_specs=[
            pl.BlockSpec((1, NT, C), lambda b: (b, 0, 0)),
            pl.BlockSpec((1, N, C), lambda b: (b, 0, 0)),
        ],
        out_specs=pl.BlockSpec((1, 1, K), lambda b: (b, 0, 0)),
        out_shape=jax.ShapeDtypeStruct((B, 1, K), jnp.int32),
    )(x, edge_aggregation)
    xflat = jnp.reshape(x, (B * NT, C))
    pooled = _sc_gather(xflat, jnp.reshape(idx, (B, K)))
    return jnp.concatenate([x[:, 0:1], pooled], axis=1)


# speedup vs baseline: 1.0119x; 1.0119x over previous
import functools

import jax
import jax.numpy as jnp
from jax import lax
from jax.experimental import pallas as pl
from jax.experimental.pallas import tpu as pltpu
from jax.experimental.pallas import tpu_sc as plsc

B, N, C = 64, 1024, 192
K = N // 4
NT = N + 1
JC = 128


def _rank_body(x_ref, ea_ref, idx_ref):
    b = pl.program_id(0)
    nodes = x_ref[0, 1:, :]
    ea = ea_ref[0]
    m = jnp.sum(ea, axis=0, keepdims=True) * (1.0 / N)
    ci = jax.nn.sigmoid(m)
    scores = jnp.dot(nodes, jnp.reshape(ci, (C, 1)),
                     preferred_element_type=jnp.float32)
    s_row = jnp.reshape(scores, (1, N))

    jj0 = lax.broadcasted_iota(jnp.int32, (JC, N), 0)
    ii = lax.broadcasted_iota(jnp.int32, (JC, N), 1)

    rank_row = jnp.zeros((1, N), jnp.int32)
    for jc in range(N // JC):
        s_col = scores[jc * JC:(jc + 1) * JC, :]
        jj = jj0 + (jc * JC)
        before = (s_col < s_row) | ((s_col == s_row) & (jj < ii))
        rank_row = rank_row + jnp.sum(before.astype(jnp.int32), axis=0,
                                      keepdims=True)
    ranks = jnp.reshape(rank_row, (N, 1))

    p_iota = lax.broadcasted_iota(jnp.int32, (N, K), 1) + (N - K)
    ivals = lax.broadcasted_iota(jnp.int32, (N, 1), 0) + (b * NT + 1)
    idx_row = jnp.sum(jnp.where(ranks == p_iota, ivals, 0), axis=0,
                      keepdims=True)
    idx_ref[0] = idx_row


_mesh = plsc.VectorSubcoreMesh(core_axis_name="c", subcore_axis_name="s")


@functools.partial(
    pl.kernel,
    out_type=jax.ShapeDtypeStruct((B, K + 1, C), jnp.float32),
    mesh=_mesh,
    scratch_types=[
        pltpu.VMEM((K,), jnp.int32),
        pltpu.VMEM((K, C), jnp.float32),
        pltpu.VMEM((1, C), jnp.float32),
        pltpu.SemaphoreType.DMA,
    ],
    compiler_params=pltpu.CompilerParams(use_tc_tiling_on_sc=False),
)
def _sc_gather(xflat, idx_hbm, out_hbm, idx_v, rows_v, cls_v, sem):
    wid = lax.axis_index("s") * 2 + lax.axis_index("c")
    for j in range(2):
        b = wid * 2 + j
        pltpu.sync_copy(idx_hbm.at[b], idx_v)
        cp0 = pltpu.async_copy(xflat.at[idx_v.at[pl.ds(0, 128)]],
                               rows_v.at[pl.ds(0, 128)], sem)
        cp1 = pltpu.async_copy(xflat.at[idx_v.at[pl.ds(128, 128)]],
                               rows_v.at[pl.ds(128, 128)], sem)
        pltpu.sync_copy(xflat.at[pl.ds(b * NT, 1)], cls_v)
        pltpu.sync_copy(cls_v, out_hbm.at[b, pl.ds(0, 1)])
        cp0.wait()
        cp1.wait()
        pltpu.sync_copy(rows_v, out_hbm.at[b, pl.ds(1, K)])


@jax.jit
def kernel(x, edge_aggregation):
    idx = pl.pallas_call(
        _rank_body,
        grid=(B,),
        in_specs=[
            pl.BlockSpec((1, NT, C), lambda b: (b, 0, 0)),
            pl.BlockSpec((1, N, C), lambda b: (b, 0, 0)),
        ],
        out_specs=pl.BlockSpec((1, 1, K), lambda b: (b, 0, 0)),
        out_shape=jax.ShapeDtypeStruct((B, 1, K), jnp.int32),
    )(x, edge_aggregation)
    xflat = jnp.reshape(x, (B * NT, C))
    return _sc_gather(xflat, jnp.reshape(idx, (B, K)))
